# Initial kernel scaffold; baseline (speedup 1.0000x reference)
import functools

import jax
import jax.numpy as jnp
from jax import lax
from jax.experimental import pallas as pl
from jax.experimental.pallas import tpu as pltpu
from jax.experimental.pallas import tpu_sc as plsc

N = 10000
E = 160000
D = 256
H = 128
NC = 2
NS = 16
NPAD = 10240
EPC = 128
CH = 80
EPAD = NS * CH * EPC
RPT = NPAD // NS

_mesh = plsc.VectorSubcoreMesh(core_axis_name="c", subcore_axis_name="s")


@functools.partial(
    pl.kernel,
    out_type=jax.ShapeDtypeStruct((NC, NPAD), jnp.float32),
    mesh=_mesh,
    scratch_types=[
        pltpu.VMEM((CH // 2, EPC), jnp.int32),
        pltpu.VMEM((NPAD,), jnp.float32),
        pltpu.VMEM((NS, RPT), jnp.float32),
        pltpu.VMEM((RPT,), jnp.float32),
        pltpu.VMEM_SHARED((NS, NPAD), jnp.float32),
    ],
)
def _deg_kernel(dst_hbm, deg_hbm, dst_v, hist, part, seg, shared):
    c = lax.axis_index("c")
    s = lax.axis_index("s")
    nch = CH // 2
    pltpu.sync_copy(dst_hbm.at[s, pl.ds(c * nch, nch)], dst_v)

    def _zero(i, _):
        hist[pl.ds(i * 16, 16)] = jnp.zeros((16,), jnp.float32)
        return 0
    lax.fori_loop(0, NPAD // 16, _zero, 0)

    ones = jnp.ones((16,), jnp.float32)

    def _count(j, _):
        for t in range(EPC // 16):
            idx = dst_v[j, pl.ds(t * 16, 16)]
            plsc.addupdate_scatter(hist, [idx], ones)
        return 0
    lax.fori_loop(0, nch, _count, 0)

    pltpu.sync_copy(hist, shared.at[s])
    plsc.subcore_barrier()
    pltpu.sync_copy(shared.at[:, pl.ds(s * RPT, RPT)], part)

    def _sum(v, _):
        t = jnp.zeros((16,), jnp.float32)
        for r in range(NS):
            t = t + part[r, pl.ds(v * 16, 16)]
        seg[pl.ds(v * 16, 16)] = t
        return 0
    lax.fori_loop(0, RPT // 16, _sum, 0)
    pltpu.sync_copy(seg, deg_hbm.at[c, pl.ds(s * RPT, RPT)])


@functools.partial(
    pl.kernel,
    out_type=jax.ShapeDtypeStruct((NC, NPAD, H), jnp.float32),
    mesh=_mesh,
    scratch_types=[
        pltpu.VMEM((CH, EPC), jnp.int32),
        pltpu.VMEM((CH, EPC), jnp.int32),
        pltpu.VMEM((EPC, H), jnp.float32),
        pltpu.VMEM_SHARED((NPAD, H), jnp.float32),
        pltpu.SemaphoreType.DMA,
    ],
)
def _agg_kernel(g_hbm, srcg_hbm, dst_hbm, zeros_hbm, agg_hbm,
                src_v, dst_v, buf, acc, sem):
    c = lax.axis_index("c")
    s = lax.axis_index("s")
    pltpu.sync_copy(srcg_hbm.at[c, s], src_v)
    pltpu.sync_copy(dst_hbm.at[s], dst_v)
    pltpu.sync_copy(zeros_hbm, acc.at[pl.ds(s * RPT, RPT)])
    plsc.subcore_barrier()

    def _edge(j, _):
        pltpu.async_copy(g_hbm.at[src_v.at[j]], buf, sem).wait()
        pltpu.sync_copy(buf, acc.at[dst_v.at[j]], add=True)
        return 0
    lax.fori_loop(0, CH, _edge, 0)

    plsc.subcore_barrier()
    pltpu.sync_copy(acc.at[pl.ds(s * RPT, RPT)],
                    agg_hbm.at[c].at[pl.ds(s * RPT, RPT)])


_RB = 1024


def _tc1_body(x_ref, w_ref, deg_ref, g_ref, dinv_ref):
    deg = 1.0 + deg_ref[0] + deg_ref[1]
    dinv = lax.rsqrt(deg)
    h = jnp.dot(x_ref[...], w_ref[...], preferred_element_type=jnp.float32)
    g = h * dinv
    g_ref[...] = jnp.stack([g[:, :H], g[:, H:]], axis=0)
    dinv_ref[...] = dinv


def _tc1(x, w0, degpart):
    return pl.pallas_call(
        _tc1_body,
        grid=(NPAD // _RB,),
        in_specs=[
            pl.BlockSpec((_RB, D), lambda i: (i, 0)),
            pl.BlockSpec((D, D), lambda i: (0, 0)),
            pl.BlockSpec((NC, _RB, 1), lambda i: (0, i, 0)),
        ],
        out_specs=[
            pl.BlockSpec((NC, _RB, H), lambda i: (0, i, 0)),
            pl.BlockSpec((_RB, 1), lambda i: (i, 0)),
        ],
        out_shape=[
            jax.ShapeDtypeStruct((NC, NPAD, H), jnp.float32),
            jax.ShapeDtypeStruct((NPAD, 1), jnp.float32),
        ],
    )(x, w0, degpart)


def _tc2_body(g_ref, agg_ref, dinv_ref, b_ref, w_ref, gout_ref):
    pre = agg_ref[...] + g_ref[...]
    hfull = jnp.concatenate([pre[0], pre[1]], axis=1)
    dinv = dinv_ref[...]
    h1 = jnp.maximum(dinv * hfull + b_ref[...], 0.0)
    m = jnp.dot(h1, w_ref[...], preferred_element_type=jnp.float32)
    g1 = m * dinv
    gout_ref[...] = jnp.stack([g1[:, :H], g1[:, H:]], axis=0)


def _tc2(g0, agg0, dinv, b0, w1):
    return pl.pallas_call(
        _tc2_body,
        grid=(NPAD // _RB,),
        in_specs=[
            pl.BlockSpec((NC, _RB, H), lambda i: (0, i, 0)),
            pl.BlockSpec((NC, _RB, H), lambda i: (0, i, 0)),
            pl.BlockSpec((_RB, 1), lambda i: (i, 0)),
            pl.BlockSpec((1, D), lambda i: (0, 0)),
            pl.BlockSpec((D, D), lambda i: (0, 0)),
        ],
        out_specs=pl.BlockSpec((NC, _RB, H), lambda i: (0, i, 0)),
        out_shape=jax.ShapeDtypeStruct((NC, NPAD, H), jnp.float32),
    )(g0, agg0, dinv, b0, w1)


def _tc3_body(g_ref, agg_ref, dinv_ref, b_ref, out_ref):
    pre = agg_ref[...] + g_ref[...]
    hfull = jnp.concatenate([pre[0], pre[1]], axis=1)
    out_ref[...] = dinv_ref[...] * hfull + b_ref[...]


def _tc3(g1, agg1, dinv, b1):
    return pl.pallas_call(
        _tc3_body,
        grid=(NPAD // _RB,),
        in_specs=[
            pl.BlockSpec((NC, _RB, H), lambda i: (0, i, 0)),
            pl.BlockSpec((NC, _RB, H), lambda i: (0, i, 0)),
            pl.BlockSpec((_RB, 1), lambda i: (i, 0)),
            pl.BlockSpec((1, D), lambda i: (0, 0)),
        ],
        out_specs=pl.BlockSpec((_RB, D), lambda i: (i, 0)),
        out_shape=jax.ShapeDtypeStruct((NPAD, D), jnp.float32),
    )(g1, agg1, dinv, b1)


def kernel(x, edge_index, W0, b0, W1, b1):
    src = edge_index[0].astype(jnp.int32)
    dst = edge_index[1].astype(jnp.int32)
    fill = jnp.full((EPAD - E,), N, dtype=jnp.int32)
    src_p = jnp.concatenate([src, fill])
    dst_p = jnp.concatenate([dst, fill])
    srcg = (src_p[None, :] + jnp.arange(NC, dtype=jnp.int32)[:, None] * NPAD
            ).reshape(NC, NS, CH, EPC)
    dsts = dst_p.reshape(NS, CH, EPC)

    x_pad = jnp.pad(x, ((0, NPAD - N), (0, 0)))
    zeros = jnp.zeros((RPT, H), jnp.float32)

    degpart = _deg_kernel(dsts)
    g0, dinv = _tc1(x_pad, W0, degpart.reshape(NC, NPAD, 1))
    agg0 = _agg_kernel(g0.reshape(NC * NPAD, H), srcg, dsts, zeros)
    g1 = _tc2(g0, agg0, dinv, b0.reshape(1, D), W1)
    agg1 = _agg_kernel(g1.reshape(NC * NPAD, H), srcg, dsts, zeros)
    out = _tc3(g1, agg1, dinv, b1.reshape(1, D))
    return out[:N]


# speedup vs baseline: 7.2516x; 7.2516x over previous
import functools

import jax
import jax.numpy as jnp
from jax import lax
from jax.experimental import pallas as pl
from jax.experimental.pallas import tpu as pltpu
from jax.experimental.pallas import tpu_sc as plsc

N = 10000
E = 160000
D = 256
H = 128
NC = 2
NS = 16
NPAD = 10240
EPC = 128
CH = 80
EPAD = NS * CH * EPC
RPT = NPAD // NS

_mesh = plsc.VectorSubcoreMesh(core_axis_name="c", subcore_axis_name="s")


@functools.partial(
    pl.kernel,
    out_type=jax.ShapeDtypeStruct((NC, NPAD), jnp.float32),
    mesh=_mesh,
    scratch_types=[
        pltpu.VMEM((CH // 2, EPC), jnp.int32),
        pltpu.VMEM((NPAD,), jnp.float32),
        pltpu.VMEM((NS, RPT), jnp.float32),
        pltpu.VMEM((RPT,), jnp.float32),
        pltpu.VMEM_SHARED((NS, NPAD), jnp.float32),
    ],
    compiler_params=pltpu.CompilerParams(needs_layout_passes=False),
)
def _deg_kernel(dst_hbm, deg_hbm, dst_v, hist, part, seg, shared):
    c = lax.axis_index("c")
    s = lax.axis_index("s")
    nch = CH // 2
    pltpu.sync_copy(dst_hbm.at[s, pl.ds(c * nch, nch)], dst_v)

    def _zero(i, _):
        hist[pl.ds(i * 16, 16)] = jnp.zeros((16,), jnp.float32)
        return 0
    lax.fori_loop(0, NPAD // 16, _zero, 0)

    ones = jnp.ones((16,), jnp.float32)

    def _count(j, _):
        for t in range(EPC // 16):
            idx = dst_v[j, pl.ds(t * 16, 16)]
            plsc.addupdate_scatter(hist, [idx], ones)
        return 0
    lax.fori_loop(0, nch, _count, 0)

    pltpu.sync_copy(hist, shared.at[s])
    plsc.subcore_barrier()
    pltpu.sync_copy(shared.at[:, pl.ds(s * RPT, RPT)], part)

    def _sum(v, _):
        t = jnp.zeros((16,), jnp.float32)
        for r in range(NS):
            t = t + part[r, pl.ds(v * 16, 16)]
        seg[pl.ds(v * 16, 16)] = t
        return 0
    lax.fori_loop(0, RPT // 16, _sum, 0)
    pltpu.sync_copy(seg, deg_hbm.at[c, pl.ds(s * RPT, RPT)])


@functools.partial(
    pl.kernel,
    out_type=jax.ShapeDtypeStruct((NC, NPAD, H), jnp.float32),
    mesh=_mesh,
    scratch_types=[
        pltpu.VMEM((CH, EPC), jnp.int32),
        pltpu.VMEM((CH, EPC), jnp.int32),
        pltpu.VMEM((EPC, H), jnp.float32),
        pltpu.VMEM_SHARED((NPAD, H), jnp.float32),
        pltpu.SemaphoreType.DMA,
    ],
    compiler_params=pltpu.CompilerParams(needs_layout_passes=False),
)
def _agg_kernel(g_hbm, srcg_hbm, dst_hbm, zeros_hbm, agg_hbm,
                src_v, dst_v, buf, acc, sem):
    c = lax.axis_index("c")
    s = lax.axis_index("s")
    pltpu.sync_copy(srcg_hbm.at[c, s], src_v)
    pltpu.sync_copy(dst_hbm.at[s], dst_v)
    pltpu.sync_copy(zeros_hbm, acc.at[pl.ds(s * RPT, RPT)])
    plsc.subcore_barrier()

    def _edge(j, _):
        pltpu.async_copy(g_hbm.at[src_v.at[j]], buf, sem).wait()
        pltpu.sync_copy(buf, acc.at[dst_v.at[j]], add=True)
        return 0
    lax.fori_loop(0, CH, _edge, 0)

    plsc.subcore_barrier()
    pltpu.sync_copy(acc.at[pl.ds(s * RPT, RPT)],
                    agg_hbm.at[c].at[pl.ds(s * RPT, RPT)])


_RB = 1024


def _tc1_body(x_ref, w_ref, deg_ref, g_ref, dinv_ref):
    deg = 1.0 + deg_ref[0] + deg_ref[1]
    dinv = lax.rsqrt(deg)
    h = jnp.dot(x_ref[...], w_ref[...], preferred_element_type=jnp.float32)
    g = h * dinv
    g_ref[...] = jnp.stack([g[:, :H], g[:, H:]], axis=0)
    dinv_ref[...] = dinv


def _tc1(x, w0, degpart):
    return pl.pallas_call(
        _tc1_body,
        grid=(NPAD // _RB,),
        in_specs=[
            pl.BlockSpec((_RB, D), lambda i: (i, 0)),
            pl.BlockSpec((D, D), lambda i: (0, 0)),
            pl.BlockSpec((NC, _RB, 1), lambda i: (0, i, 0)),
        ],
        out_specs=[
            pl.BlockSpec((NC, _RB, H), lambda i: (0, i, 0)),
            pl.BlockSpec((_RB, 1), lambda i: (i, 0)),
        ],
        out_shape=[
            jax.ShapeDtypeStruct((NC, NPAD, H), jnp.float32),
            jax.ShapeDtypeStruct((NPAD, 1), jnp.float32),
        ],
    )(x, w0, degpart)


def _tc2_body(g_ref, agg_ref, dinv_ref, b_ref, w_ref, gout_ref):
    pre = agg_ref[...] + g_ref[...]
    hfull = jnp.concatenate([pre[0], pre[1]], axis=1)
    dinv = dinv_ref[...]
    h1 = jnp.maximum(dinv * hfull + b_ref[...], 0.0)
    m = jnp.dot(h1, w_ref[...], preferred_element_type=jnp.float32)
    g1 = m * dinv
    gout_ref[...] = jnp.stack([g1[:, :H], g1[:, H:]], axis=0)


def _tc2(g0, agg0, dinv, b0, w1):
    return pl.pallas_call(
        _tc2_body,
        grid=(NPAD // _RB,),
        in_specs=[
            pl.BlockSpec((NC, _RB, H), lambda i: (0, i, 0)),
            pl.BlockSpec((NC, _RB, H), lambda i: (0, i, 0)),
            pl.BlockSpec((_RB, 1), lambda i: (i, 0)),
            pl.BlockSpec((1, D), lambda i: (0, 0)),
            pl.BlockSpec((D, D), lambda i: (0, 0)),
        ],
        out_specs=pl.BlockSpec((NC, _RB, H), lambda i: (0, i, 0)),
        out_shape=jax.ShapeDtypeStruct((NC, NPAD, H), jnp.float32),
    )(g0, agg0, dinv, b0, w1)


def _tc3_body(g_ref, agg_ref, dinv_ref, b_ref, out_ref):
    pre = agg_ref[...] + g_ref[...]
    hfull = jnp.concatenate([pre[0], pre[1]], axis=1)
    out_ref[...] = dinv_ref[...] * hfull + b_ref[...]


def _tc3(g1, agg1, dinv, b1):
    return pl.pallas_call(
        _tc3_body,
        grid=(NPAD // _RB,),
        in_specs=[
            pl.BlockSpec((NC, _RB, H), lambda i: (0, i, 0)),
            pl.BlockSpec((NC, _RB, H), lambda i: (0, i, 0)),
            pl.BlockSpec((_RB, 1), lambda i: (i, 0)),
            pl.BlockSpec((1, D), lambda i: (0, 0)),
        ],
        out_specs=pl.BlockSpec((_RB, D), lambda i: (i, 0)),
        out_shape=jax.ShapeDtypeStruct((NPAD, D), jnp.float32),
    )(g1, agg1, dinv, b1)


def kernel(x, edge_index, W0, b0, W1, b1):
    src = edge_index[0].astype(jnp.int32)
    dst = edge_index[1].astype(jnp.int32)
    fill = jnp.full((EPAD - E,), N, dtype=jnp.int32)
    src_p = jnp.concatenate([src, fill])
    dst_p = jnp.concatenate([dst, fill])
    srcg = (src_p[None, :] + jnp.arange(NC, dtype=jnp.int32)[:, None] * NPAD
            ).reshape(NC, NS, CH, EPC)
    dsts = dst_p.reshape(NS, CH, EPC)

    x_pad = jnp.pad(x, ((0, NPAD - N), (0, 0)))
    zeros = jnp.zeros((RPT, H), jnp.float32)

    degpart = _deg_kernel(dsts)
    g0, dinv = _tc1(x_pad, W0, degpart.reshape(NC, NPAD, 1))
    agg0 = _agg_kernel(g0.reshape(NC * NPAD, H), srcg, dsts, zeros)
    g1 = _tc2(g0, agg0, dinv, b0.reshape(1, D), W1)
    agg1 = _agg_kernel(g1.reshape(NC * NPAD, H), srcg, dsts, zeros)
    out = _tc3(g1, agg1, dinv, b1.reshape(1, D))
    return out[:N]


# speedup vs baseline: 8.7473x; 1.2063x over previous
import functools

import jax
import jax.numpy as jnp
from jax import lax
from jax.experimental import pallas as pl
from jax.experimental.pallas import tpu as pltpu
from jax.experimental.pallas import tpu_sc as plsc

N = 10000
E = 160000
D = 256
H = 128
NC = 2
NS = 16
NPAD = 10240
EPC = 128
CH = 80
EPAD = NS * CH * EPC
RPT = NPAD // NS

_mesh = plsc.VectorSubcoreMesh(core_axis_name="c", subcore_axis_name="s")


@functools.partial(
    pl.kernel,
    out_type=jax.ShapeDtypeStruct((NC, NPAD), jnp.float32),
    mesh=_mesh,
    scratch_types=[
        pltpu.VMEM((CH // 2, EPC), jnp.int32),
        pltpu.VMEM((NPAD,), jnp.float32),
        pltpu.VMEM((NS, RPT), jnp.float32),
        pltpu.VMEM((RPT,), jnp.float32),
        pltpu.VMEM_SHARED((NS, NPAD), jnp.float32),
    ],
    compiler_params=pltpu.CompilerParams(needs_layout_passes=False),
)
def _deg_kernel(dst_hbm, deg_hbm, dst_v, hist, part, seg, shared):
    c = lax.axis_index("c")
    s = lax.axis_index("s")
    nch = CH // 2
    pltpu.sync_copy(dst_hbm.at[s, pl.ds(c * nch, nch)], dst_v)

    def _zero(i, _):
        hist[pl.ds(i * 16, 16)] = jnp.zeros((16,), jnp.float32)
        return 0
    lax.fori_loop(0, NPAD // 16, _zero, 0)

    ones = jnp.ones((16,), jnp.float32)

    def _count(j, _):
        for t in range(EPC // 16):
            idx = dst_v[j, pl.ds(t * 16, 16)]
            plsc.addupdate_scatter(hist, [idx], ones)
        return 0
    lax.fori_loop(0, nch, _count, 0)

    pltpu.sync_copy(hist, shared.at[s])
    plsc.subcore_barrier()
    pltpu.sync_copy(shared.at[:, pl.ds(s * RPT, RPT)], part)

    def _sum(v, _):
        t = jnp.zeros((16,), jnp.float32)
        for r in range(NS):
            t = t + part[r, pl.ds(v * 16, 16)]
        seg[pl.ds(v * 16, 16)] = t
        return 0
    lax.fori_loop(0, RPT // 16, _sum, 0)
    pltpu.sync_copy(seg, deg_hbm.at[c, pl.ds(s * RPT, RPT)])


@functools.partial(
    pl.kernel,
    out_type=jax.ShapeDtypeStruct((NC, NPAD, H), jnp.float32),
    mesh=_mesh,
    scratch_types=[
        [pltpu.VMEM((EPC,), jnp.int32)] * 4,
        pltpu.VMEM((CH, EPC), jnp.int32),
        [pltpu.VMEM((EPC, H), jnp.float32)] * 2,
        pltpu.VMEM_SHARED((NPAD, H), jnp.float32),
        [pltpu.SemaphoreType.DMA] * 4,
        [pltpu.SemaphoreType.DMA] * 2,
    ],
    compiler_params=pltpu.CompilerParams(needs_layout_passes=False),
)
def _agg_kernel(g_hbm, srcg_hbm, dst_hbm, zeros_hbm, agg_hbm,
                src_rg, dst_v, bufs, acc, isems, gsems):
    c = lax.axis_index("c")
    s = lax.axis_index("s")
    pltpu.sync_copy(dst_hbm.at[s], dst_v)
    pltpu.sync_copy(zeros_hbm, acc.at[pl.ds(s * RPT, RPT)])

    NI, NB = 4, 2
    for m in range(NI):
        pltpu.async_copy(srcg_hbm.at[c, s, m], src_rg[m], isems[m])
    plsc.subcore_barrier()
    for b in range(NB):
        pltpu.make_async_copy(srcg_hbm.at[c, s, b], src_rg[b],
                              isems[b]).wait()
        pltpu.async_copy(g_hbm.at[src_rg[b]], bufs[b], gsems[b])

    def _step(u, _):
        for k in range(NI):
            b = k % NB
            j = u * NI + k
            pltpu.make_async_copy(g_hbm.at[src_rg[b]], bufs[b],
                                  gsems[b]).wait()
            pltpu.sync_copy(bufs[b], acc.at[dst_v.at[j]], add=True)

            @pl.when(j + NB < CH)
            def _issue_gather():
                m = (k + NB) % NI
                pltpu.make_async_copy(srcg_hbm.at[c, s, j + NB], src_rg[m],
                                      isems[m]).wait()
                pltpu.async_copy(g_hbm.at[src_rg[m]], bufs[b], gsems[b])

            @pl.when(j + NI < CH)
            def _issue_idx():
                pltpu.async_copy(srcg_hbm.at[c, s, j + NI], src_rg[k],
                                 isems[k])
        return 0
    lax.fori_loop(0, CH // NI, _step, 0)

    plsc.subcore_barrier()
    pltpu.sync_copy(acc.at[pl.ds(s * RPT, RPT)],
                    agg_hbm.at[c].at[pl.ds(s * RPT, RPT)])


_RB = 1024


def _tc1_body(x_ref, w_ref, deg_ref, g_ref, dinv_ref):
    deg = 1.0 + deg_ref[0] + deg_ref[1]
    dinv = lax.rsqrt(deg)
    h = jnp.dot(x_ref[...], w_ref[...], preferred_element_type=jnp.float32)
    g = h * dinv
    g_ref[...] = jnp.stack([g[:, :H], g[:, H:]], axis=0)
    dinv_ref[...] = dinv


def _tc1(x, w0, degpart):
    return pl.pallas_call(
        _tc1_body,
        grid=(NPAD // _RB,),
        in_specs=[
            pl.BlockSpec((_RB, D), lambda i: (i, 0)),
            pl.BlockSpec((D, D), lambda i: (0, 0)),
            pl.BlockSpec((NC, _RB, 1), lambda i: (0, i, 0)),
        ],
        out_specs=[
            pl.BlockSpec((NC, _RB, H), lambda i: (0, i, 0)),
            pl.BlockSpec((_RB, 1), lambda i: (i, 0)),
        ],
        out_shape=[
            jax.ShapeDtypeStruct((NC, NPAD, H), jnp.float32),
            jax.ShapeDtypeStruct((NPAD, 1), jnp.float32),
        ],
    )(x, w0, degpart)


def _tc2_body(g_ref, agg_ref, dinv_ref, b_ref, w_ref, gout_ref):
    pre = agg_ref[...] + g_ref[...]
    hfull = jnp.concatenate([pre[0], pre[1]], axis=1)
    dinv = dinv_ref[...]
    h1 = jnp.maximum(dinv * hfull + b_ref[...], 0.0)
    m = jnp.dot(h1, w_ref[...], preferred_element_type=jnp.float32)
    g1 = m * dinv
    gout_ref[...] = jnp.stack([g1[:, :H], g1[:, H:]], axis=0)


def _tc2(g0, agg0, dinv, b0, w1):
    return pl.pallas_call(
        _tc2_body,
        grid=(NPAD // _RB,),
        in_specs=[
            pl.BlockSpec((NC, _RB, H), lambda i: (0, i, 0)),
            pl.BlockSpec((NC, _RB, H), lambda i: (0, i, 0)),
            pl.BlockSpec((_RB, 1), lambda i: (i, 0)),
            pl.BlockSpec((1, D), lambda i: (0, 0)),
            pl.BlockSpec((D, D), lambda i: (0, 0)),
        ],
        out_specs=pl.BlockSpec((NC, _RB, H), lambda i: (0, i, 0)),
        out_shape=jax.ShapeDtypeStruct((NC, NPAD, H), jnp.float32),
    )(g0, agg0, dinv, b0, w1)


def _tc3_body(g_ref, agg_ref, dinv_ref, b_ref, out_ref):
    pre = agg_ref[...] + g_ref[...]
    hfull = jnp.concatenate([pre[0], pre[1]], axis=1)
    out_ref[...] = dinv_ref[...] * hfull + b_ref[...]


def _tc3(g1, agg1, dinv, b1):
    return pl.pallas_call(
        _tc3_body,
        grid=(NPAD // _RB,),
        in_specs=[
            pl.BlockSpec((NC, _RB, H), lambda i: (0, i, 0)),
            pl.BlockSpec((NC, _RB, H), lambda i: (0, i, 0)),
            pl.BlockSpec((_RB, 1), lambda i: (i, 0)),
            pl.BlockSpec((1, D), lambda i: (0, 0)),
        ],
        out_specs=pl.BlockSpec((_RB, D), lambda i: (i, 0)),
        out_shape=jax.ShapeDtypeStruct((NPAD, D), jnp.float32),
    )(g1, agg1, dinv, b1)


def kernel(x, edge_index, W0, b0, W1, b1):
    src = edge_index[0].astype(jnp.int32)
    dst = edge_index[1].astype(jnp.int32)
    fill = jnp.full((EPAD - E,), N, dtype=jnp.int32)
    src_p = jnp.concatenate([src, fill])
    dst_p = jnp.concatenate([dst, fill])
    srcg = (src_p[None, :] + jnp.arange(NC, dtype=jnp.int32)[:, None] * NPAD
            ).reshape(NC, NS, CH, EPC)
    dsts = dst_p.reshape(NS, CH, EPC)

    x_pad = jnp.pad(x, ((0, NPAD - N), (0, 0)))
    zeros = jnp.zeros((RPT, H), jnp.float32)

    degpart = _deg_kernel(dsts)
    g0, dinv = _tc1(x_pad, W0, degpart.reshape(NC, NPAD, 1))
    agg0 = _agg_kernel(g0.reshape(NC * NPAD, H), srcg, dsts, zeros)
    g1 = _tc2(g0, agg0, dinv, b0.reshape(1, D), W1)
    agg1 = _agg_kernel(g1.reshape(NC * NPAD, H), srcg, dsts, zeros)
    out = _tc3(g1, agg1, dinv, b1.reshape(1, D))
    return out[:N]
